# Initial kernel scaffold; baseline (speedup 1.0000x reference)
import jax
import jax.numpy as jnp
from jax.experimental import pallas as pl


def kernel(x, edge_index, batch, aW1, ab1, aW2, ab2, mW, mb, log_std, cW1, cb1, cW2, cb2, fW1, fb1, fW2, fb2):
    raise NotImplementedError("write your pallas kernel here")


# speedup vs baseline: 20.4704x; 20.4704x over previous
import functools

import jax
import jax.numpy as jnp
from jax import lax
from jax.experimental import pallas as pl
from jax.experimental.pallas import tpu as pltpu
from jax.experimental.pallas import tpu_sc as plsc

N = 10000
NP = 10240
D = 128
H = 64
E = 320000
EP = 327680
ROWS_E = EP // 128
CH_A = EP // (32 * 128)
CH_B = EP // (16 * 128)
RPT = NP // 16
G = 64
BM = 2048
GRID = NP // BM

_f32 = jnp.float32
_SC_PARAMS = pltpu.CompilerParams(use_tc_tiling_on_sc=False)


def _sc_mesh():
    return plsc.VectorSubcoreMesh(core_axis_name="c", subcore_axis_name="s")


def _deg_body(dst_hbm, out_hbm, idx_v, ones_v, bounce_v, acc_sh):
    c = lax.axis_index("c")
    s = lax.axis_index("s")
    one16 = jnp.ones((16,), _f32)
    zero16 = jnp.zeros((16,), _f32)

    def fill(i, carry):
        ones_v[i] = one16
        bounce_v[i] = zero16
        return carry

    lax.fori_loop(0, 128, fill, 0)
    r0 = s * RPT
    for k in range(RPT // 128):
        pltpu.sync_copy(bounce_v, acc_sh.at[pl.ds(r0 + k * 128, 128)])
    plsc.subcore_barrier()

    w = c * 16 + s
    pltpu.sync_copy(dst_hbm.at[pl.ds(w * CH_A, CH_A)], idx_v)

    def chunk(j, carry):
        pltpu.sync_copy(ones_v, acc_sh.at[idx_v.at[j]], add=True)
        return carry

    lax.fori_loop(0, CH_A, chunk, 0)
    plsc.subcore_barrier()
    for k in range(RPT // 128):
        pltpu.sync_copy(acc_sh.at[pl.ds(r0 + k * 128, 128)], bounce_v)
        pltpu.sync_copy(bounce_v, out_hbm.at[c, pl.ds(r0 + k * 128, 128)])


_deg_call = pl.kernel(
    _deg_body,
    out_type=jax.ShapeDtypeStruct((2, NP, 16), _f32),
    mesh=_sc_mesh(),
    compiler_params=_SC_PARAMS,
    scratch_types=[
        pltpu.VMEM((CH_A, 128), jnp.int32),
        pltpu.VMEM((128, 16), _f32),
        pltpu.VMEM((128, 16), _f32),
        pltpu.VMEM_SHARED((NP, 16), _f32),
    ],
)


def _edge_body(t0, t1, src_hbm, dst_hbm, out0, out1,
               srcv, dstv, rows0, rows1, zv, acc_sh, sem0, sem1):
    c = lax.axis_index("c")
    s = lax.axis_index("s")
    zero16 = jnp.zeros((16,), _f32)

    def zfill(i, carry):
        for k in range(H // 16):
            zv[i, pl.ds(k * 16, 16)] = zero16
        return carry

    lax.fori_loop(0, 128, zfill, 0)
    r0 = s * RPT
    for k in range(RPT // 128):
        pltpu.sync_copy(zv, acc_sh.at[pl.ds(r0 + k * 128, 128)])
    plsc.subcore_barrier()

    pltpu.sync_copy(src_hbm.at[pl.ds(s * CH_B, CH_B)], srcv)
    pltpu.sync_copy(dst_hbm.at[pl.ds(s * CH_B, CH_B)], dstv)

    bufs = (rows0, rows1)
    sems = (sem0, sem1)
    for cid, y_cid, out_cid in ((0, t0, out0), (1, t1, out1)):
        @pl.when(c == cid)
        def _(y_hbm=y_cid, out_hbm=out_cid):
            pltpu.make_async_copy(y_hbm.at[srcv.at[0]], rows0, sem0).start()
            pltpu.make_async_copy(y_hbm.at[srcv.at[1]], rows1, sem1).start()

            def loop(j2, carry):
                for b in range(2):
                    j = 2 * j2 + b
                    pltpu.make_async_copy(y_hbm.at[srcv.at[j]], bufs[b], sems[b]).wait()
                    pltpu.sync_copy(bufs[b], acc_sh.at[dstv.at[j]], add=True)

                    @pl.when(j + 2 < CH_B)
                    def _fire(j=j, b=b):
                        pltpu.make_async_copy(
                            y_hbm.at[srcv.at[j + 2]], bufs[b], sems[b]).start()
                return carry

            lax.fori_loop(0, CH_B // 2, loop, 0)
            plsc.subcore_barrier()
            for k in range(RPT // 128):
                pltpu.sync_copy(acc_sh.at[pl.ds(r0 + k * 128, 128)], zv)
                pltpu.sync_copy(zv, out_hbm.at[pl.ds(r0 + k * 128, 128)])


_edge_call = pl.kernel(
    _edge_body,
    out_type=[jax.ShapeDtypeStruct((NP, H), _f32)] * 2,
    mesh=_sc_mesh(),
    compiler_params=_SC_PARAMS,
    scratch_types=[
        pltpu.VMEM((CH_B, 128), jnp.int32),
        pltpu.VMEM((CH_B, 128), jnp.int32),
        pltpu.VMEM((128, H), _f32),
        pltpu.VMEM((128, H), _f32),
        pltpu.VMEM((128, H), _f32),
        pltpu.VMEM_SHARED((NP, H), _f32),
        pltpu.SemaphoreType.DMA,
        pltpu.SemaphoreType.DMA,
    ],
)


def _dinv(deg_ref):
    da = deg_ref[0, :, 0:1]
    db = deg_ref[1, :, 0:1]
    return lax.rsqrt(da + db + 1.0)


def _prep_body(x_ref, deg_ref, w_ref, o0, o1, o2, o3):
    dinv = _dinv(deg_ref)
    h = jnp.dot(x_ref[...], w_ref[...], preferred_element_type=_f32)
    y = h * dinv
    o0[...] = y[:, 0 * H:1 * H]
    o1[...] = y[:, 1 * H:2 * H]
    o2[...] = y[:, 2 * H:3 * H]
    o3[...] = y[:, 3 * H:4 * H]


_prep = pl.pallas_call(
    _prep_body,
    grid=(GRID,),
    in_specs=[
        pl.BlockSpec((BM, D), lambda i: (i, 0)),
        pl.BlockSpec((2, BM, 16), lambda i: (0, i, 0)),
        pl.BlockSpec((D, 2 * D), lambda i: (0, 0)),
    ],
    out_specs=[pl.BlockSpec((BM, H), lambda i: (i, 0))] * 4,
    out_shape=[jax.ShapeDtypeStruct((NP, H), _f32)] * 4,
)


def _mid_body(aal, aah, acl, ach, yal, yah, ycl, ych, deg_ref,
              aW2, cW2, ab1, cb1, o0, o1, o2, o3):
    dinv = _dinv(deg_ref)
    za = jnp.concatenate([aal[...] + yal[...], aah[...] + yah[...]], axis=1)
    ha = jax.nn.relu(dinv * za + ab1[...])
    y2a = dinv * jnp.dot(ha, aW2[...], preferred_element_type=_f32)
    zc = jnp.concatenate([acl[...] + ycl[...], ach[...] + ych[...]], axis=1)
    hc = jax.nn.relu(dinv * zc + cb1[...])
    y2c = dinv * jnp.dot(hc, cW2[...], preferred_element_type=_f32)
    o0[...] = y2a[:, :H]
    o1[...] = y2a[:, H:]
    o2[...] = y2c[:, :H]
    o3[...] = y2c[:, H:]


_mid = pl.pallas_call(
    _mid_body,
    grid=(GRID,),
    in_specs=[pl.BlockSpec((BM, H), lambda i: (i, 0))] * 8 + [
        pl.BlockSpec((2, BM, 16), lambda i: (0, i, 0)),
        pl.BlockSpec((D, D), lambda i: (0, 0)),
        pl.BlockSpec((D, D), lambda i: (0, 0)),
        pl.BlockSpec((1, D), lambda i: (0, 0)),
        pl.BlockSpec((1, D), lambda i: (0, 0)),
    ],
    out_specs=[pl.BlockSpec((BM, H), lambda i: (i, 0))] * 4,
    out_shape=[jax.ShapeDtypeStruct((NP, H), _f32)] * 4,
)


def _head_body(aal, aah, acl, ach, yal, yah, ycl, ych, deg_ref, batch_ref,
               ab2, cb2, mW, mb, lstd, fW1, fb1, fW2, fb2,
               mean_o, std_o, val_o, sums_sc, counts_sc):
    i = pl.program_id(0)
    dinv = _dinv(deg_ref)
    za = jnp.concatenate([aal[...] + yal[...], aah[...] + yah[...]], axis=1)
    ha = jax.nn.relu(dinv * za + ab2[...])
    mean_o[...] = jnp.tanh(
        jnp.dot(ha, mW[...], preferred_element_type=_f32) + mb[...])
    std_o[...] = jnp.broadcast_to(jnp.exp(lstd[...]), (BM, 8))

    zc = jnp.concatenate([acl[...] + ycl[...], ach[...] + ych[...]], axis=1)
    hc = jax.nn.relu(dinv * zc + cb2[...])
    b = batch_ref[...][:, 0:1]
    cols = lax.broadcasted_iota(jnp.int32, (BM, G), 1)
    oh = (b == cols).astype(_f32)

    @pl.when(i == 0)
    def _():
        sums_sc[...] = jnp.zeros_like(sums_sc)
        counts_sc[...] = jnp.zeros_like(counts_sc)

    dn = (((0,), (0,)), ((), ()))
    sums_sc[...] += lax.dot_general(oh, hc, dn, preferred_element_type=_f32)
    counts_sc[...] += lax.dot_general(
        oh, jnp.ones((BM, 8), _f32), dn, preferred_element_type=_f32)

    cnt = jnp.maximum(counts_sc[:, 0:1], 1.0)
    gm = sums_sc[...] / cnt
    hid = jax.nn.relu(jnp.dot(gm, fW1[...], preferred_element_type=_f32) + fb1[...])
    val_o[...] = jnp.dot(hid, fW2[...], preferred_element_type=_f32) + fb2[...]


_head = pl.pallas_call(
    _head_body,
    grid=(GRID,),
    in_specs=[pl.BlockSpec((BM, H), lambda i: (i, 0))] * 8 + [
        pl.BlockSpec((2, BM, 16), lambda i: (0, i, 0)),
        pl.BlockSpec((BM, 8), lambda i: (i, 0)),
        pl.BlockSpec((1, D), lambda i: (0, 0)),
        pl.BlockSpec((1, D), lambda i: (0, 0)),
        pl.BlockSpec((D, 8), lambda i: (0, 0)),
        pl.BlockSpec((1, 8), lambda i: (0, 0)),
        pl.BlockSpec((1, 8), lambda i: (0, 0)),
        pl.BlockSpec((D, G), lambda i: (0, 0)),
        pl.BlockSpec((1, G), lambda i: (0, 0)),
        pl.BlockSpec((G, 1), lambda i: (0, 0)),
        pl.BlockSpec((1, 1), lambda i: (0, 0)),
    ],
    out_specs=[
        pl.BlockSpec((BM, 8), lambda i: (i, 0)),
        pl.BlockSpec((BM, 8), lambda i: (i, 0)),
        pl.BlockSpec((G, 1), lambda i: (0, 0)),
    ],
    out_shape=[
        jax.ShapeDtypeStruct((NP, 8), _f32),
        jax.ShapeDtypeStruct((NP, 8), _f32),
        jax.ShapeDtypeStruct((G, 1), _f32),
    ],
    scratch_shapes=[
        pltpu.VMEM((G, D), _f32),
        pltpu.VMEM((G, 8), _f32),
    ],
)


def kernel(x, edge_index, batch, aW1, ab1, aW2, ab2, mW, mb, log_std,
           cW1, cb1, cW2, cb2, fW1, fb1, fW2, fb2):
    src = edge_index[0].astype(jnp.int32)
    dst = edge_index[1].astype(jnp.int32)
    pad_idx = N + (jnp.arange(EP - E, dtype=jnp.int32) % (NP - N))
    src_p = jnp.concatenate([src, pad_idx]).reshape(ROWS_E, 128)
    dst_p = jnp.concatenate([dst, pad_idx]).reshape(ROWS_E, 128)
    x_p = jnp.zeros((NP, D), _f32).at[:N].set(x)
    batch_p = jnp.full((NP, 8), G, jnp.int32).at[:N].set(
        jnp.broadcast_to(batch.astype(jnp.int32)[:, None], (N, 8)))
    W1 = jnp.concatenate([aW1, cW1], axis=1)

    deg2 = _deg_call(dst_p)
    y1al, y1ah, y1cl, y1ch = _prep(x_p, deg2, W1)
    a1al, a1cl = _edge_call(y1al, y1cl, src_p, dst_p)
    a1ah, a1ch = _edge_call(y1ah, y1ch, src_p, dst_p)
    y2al, y2ah, y2cl, y2ch = _mid(a1al, a1ah, a1cl, a1ch,
                                  y1al, y1ah, y1cl, y1ch, deg2, aW2, cW2,
                                  ab1.reshape(1, D), cb1.reshape(1, D))
    a2al, a2cl = _edge_call(y2al, y2cl, src_p, dst_p)
    a2ah, a2ch = _edge_call(y2ah, y2ch, src_p, dst_p)
    mean, std, value = _head(a2al, a2ah, a2cl, a2ch,
                             y2al, y2ah, y2cl, y2ch, deg2, batch_p,
                             ab2.reshape(1, D), cb2.reshape(1, D),
                             mW, mb.reshape(1, 8), log_std,
                             fW1, fb1.reshape(1, G), fW2, fb2.reshape(1, 1))
    return (mean[:N], std[:N], value)


# speedup vs baseline: 21.2680x; 1.0390x over previous
import functools

import jax
import jax.numpy as jnp
from jax import lax
from jax.experimental import pallas as pl
from jax.experimental.pallas import tpu as pltpu
from jax.experimental.pallas import tpu_sc as plsc

N = 10000
NP = 10240
D = 128
H = 64
E = 320000
EP = 327680
ROWS_E = EP // 128
CH_A = EP // (32 * 128)
CH_B = EP // (16 * 128)
RPT = NP // 16
G = 64
BM = 2048
GRID = NP // BM

_f32 = jnp.float32
_SC_PARAMS = pltpu.CompilerParams(use_tc_tiling_on_sc=False)


def _sc_mesh():
    return plsc.VectorSubcoreMesh(core_axis_name="c", subcore_axis_name="s")


def _deg_body(dst_hbm, out_hbm, idx_v, ones_v, bounce_v, acc_sh):
    c = lax.axis_index("c")
    s = lax.axis_index("s")
    one16 = jnp.ones((16,), _f32)
    zero16 = jnp.zeros((16,), _f32)

    def fill(i, carry):
        ones_v[i] = one16
        bounce_v[i] = zero16
        return carry

    lax.fori_loop(0, 128, fill, 0)
    r0 = s * RPT
    for k in range(RPT // 128):
        pltpu.sync_copy(bounce_v, acc_sh.at[pl.ds(r0 + k * 128, 128)])
    plsc.subcore_barrier()

    w = c * 16 + s
    pltpu.sync_copy(dst_hbm.at[pl.ds(w * CH_A, CH_A)], idx_v)

    def chunk(j, carry):
        pltpu.sync_copy(ones_v, acc_sh.at[idx_v.at[j]], add=True)
        return carry

    lax.fori_loop(0, CH_A, chunk, 0)
    plsc.subcore_barrier()
    for k in range(RPT // 128):
        pltpu.sync_copy(acc_sh.at[pl.ds(r0 + k * 128, 128)], bounce_v)
        pltpu.sync_copy(bounce_v, out_hbm.at[c, pl.ds(r0 + k * 128, 128)])


_deg_call = pl.kernel(
    _deg_body,
    out_type=jax.ShapeDtypeStruct((2, NP, 16), _f32),
    mesh=_sc_mesh(),
    compiler_params=_SC_PARAMS,
    scratch_types=[
        pltpu.VMEM((CH_A, 128), jnp.int32),
        pltpu.VMEM((128, 16), _f32),
        pltpu.VMEM((128, 16), _f32),
        pltpu.VMEM_SHARED((NP, 16), _f32),
    ],
)


_NBUF = 4


def _edge_body(t0, t1, src_hbm, dst_hbm, out0, out1,
               srcv, dstv, rows, zv, acc_sh,
               g0, g1, g2, g3, s0, s1, s2, s3):
    c = lax.axis_index("c")
    s = lax.axis_index("s")
    zero16 = jnp.zeros((16,), _f32)

    def zfill(i, carry):
        for k in range(H // 16):
            zv[i, pl.ds(k * 16, 16)] = zero16
        return carry

    lax.fori_loop(0, 128, zfill, 0)
    r0 = s * RPT
    for k in range(RPT // 128):
        pltpu.sync_copy(zv, acc_sh.at[pl.ds(r0 + k * 128, 128)])
    plsc.subcore_barrier()

    pltpu.sync_copy(src_hbm.at[pl.ds(s * CH_B, CH_B)], srcv)
    pltpu.sync_copy(dst_hbm.at[pl.ds(s * CH_B, CH_B)], dstv)

    gsems = (g0, g1, g2, g3)
    ssems = (s0, s1, s2, s3)
    for cid, y_cid, out_cid in ((0, t0, out0), (1, t1, out1)):
        @pl.when(c == cid)
        def _(y_hbm=y_cid, out_hbm=out_cid):
            pltpu.make_async_copy(y_hbm.at[srcv.at[0]], rows.at[0], gsems[0]).start()
            pltpu.make_async_copy(y_hbm.at[srcv.at[1]], rows.at[1], gsems[1]).start()

            def loop(jq, carry):
                for b in range(_NBUF):
                    j = _NBUF * jq + b
                    pltpu.make_async_copy(
                        y_hbm.at[srcv.at[j]], rows.at[b], gsems[b]).wait()
                    pltpu.async_copy(
                        rows.at[b], acc_sh.at[dstv.at[j]], ssems[b],
                        add=True)

                    @pl.when(j + 2 < CH_B)
                    def _fire(j=j, b=b):
                        b2 = (b + 2) % _NBUF

                        @pl.when(j >= 2)
                        def _drain():
                            pltpu.make_async_copy(
                                rows.at[b2],
                                acc_sh.at[dstv.at[j - 2]],
                                ssems[b2]).wait()

                        pltpu.make_async_copy(
                            y_hbm.at[srcv.at[j + 2]], rows.at[b2],
                            gsems[b2]).start()
                return carry

            lax.fori_loop(0, CH_B // _NBUF, loop, 0)
            for jj in range(CH_B - _NBUF, CH_B):
                pltpu.make_async_copy(
                    rows.at[jj % _NBUF],
                    acc_sh.at[dstv.at[jj]],
                    ssems[jj % _NBUF]).wait()
            plsc.subcore_barrier()
            for k in range(RPT // 128):
                pltpu.sync_copy(acc_sh.at[pl.ds(r0 + k * 128, 128)], zv)
                pltpu.sync_copy(zv, out_hbm.at[pl.ds(r0 + k * 128, 128)])


_edge_call = pl.kernel(
    _edge_body,
    out_type=[jax.ShapeDtypeStruct((NP, H), _f32)] * 2,
    mesh=_sc_mesh(),
    compiler_params=_SC_PARAMS,
    scratch_types=[
        pltpu.VMEM((CH_B, 128), jnp.int32),
        pltpu.VMEM((CH_B, 128), jnp.int32),
        pltpu.VMEM((_NBUF, 128, H), _f32),
        pltpu.VMEM((128, H), _f32),
        pltpu.VMEM_SHARED((NP, H), _f32),
    ] + [pltpu.SemaphoreType.DMA] * 8,
)


def _dinv(deg_ref):
    da = deg_ref[0, :, 0:1]
    db = deg_ref[1, :, 0:1]
    return lax.rsqrt(da + db + 1.0)


def _prep_body(x_ref, deg_ref, w_ref, o0, o1, o2, o3):
    dinv = _dinv(deg_ref)
    h = jnp.dot(x_ref[...], w_ref[...], preferred_element_type=_f32)
    y = h * dinv
    o0[...] = y[:, 0 * H:1 * H]
    o1[...] = y[:, 1 * H:2 * H]
    o2[...] = y[:, 2 * H:3 * H]
    o3[...] = y[:, 3 * H:4 * H]


_prep = pl.pallas_call(
    _prep_body,
    grid=(GRID,),
    in_specs=[
        pl.BlockSpec((BM, D), lambda i: (i, 0)),
        pl.BlockSpec((2, BM, 16), lambda i: (0, i, 0)),
        pl.BlockSpec((D, 2 * D), lambda i: (0, 0)),
    ],
    out_specs=[pl.BlockSpec((BM, H), lambda i: (i, 0))] * 4,
    out_shape=[jax.ShapeDtypeStruct((NP, H), _f32)] * 4,
)


def _mid_body(aal, aah, acl, ach, yal, yah, ycl, ych, deg_ref,
              aW2, cW2, ab1, cb1, o0, o1, o2, o3):
    dinv = _dinv(deg_ref)
    za = jnp.concatenate([aal[...] + yal[...], aah[...] + yah[...]], axis=1)
    ha = jax.nn.relu(dinv * za + ab1[...])
    y2a = dinv * jnp.dot(ha, aW2[...], preferred_element_type=_f32)
    zc = jnp.concatenate([acl[...] + ycl[...], ach[...] + ych[...]], axis=1)
    hc = jax.nn.relu(dinv * zc + cb1[...])
    y2c = dinv * jnp.dot(hc, cW2[...], preferred_element_type=_f32)
    o0[...] = y2a[:, :H]
    o1[...] = y2a[:, H:]
    o2[...] = y2c[:, :H]
    o3[...] = y2c[:, H:]


_mid = pl.pallas_call(
    _mid_body,
    grid=(GRID,),
    in_specs=[pl.BlockSpec((BM, H), lambda i: (i, 0))] * 8 + [
        pl.BlockSpec((2, BM, 16), lambda i: (0, i, 0)),
        pl.BlockSpec((D, D), lambda i: (0, 0)),
        pl.BlockSpec((D, D), lambda i: (0, 0)),
        pl.BlockSpec((1, D), lambda i: (0, 0)),
        pl.BlockSpec((1, D), lambda i: (0, 0)),
    ],
    out_specs=[pl.BlockSpec((BM, H), lambda i: (i, 0))] * 4,
    out_shape=[jax.ShapeDtypeStruct((NP, H), _f32)] * 4,
)


def _head_body(aal, aah, acl, ach, yal, yah, ycl, ych, deg_ref, batch_ref,
               ab2, cb2, mW, mb, lstd, fW1, fb1, fW2, fb2,
               mean_o, std_o, val_o, sums_sc, counts_sc):
    i = pl.program_id(0)
    dinv = _dinv(deg_ref)
    za = jnp.concatenate([aal[...] + yal[...], aah[...] + yah[...]], axis=1)
    ha = jax.nn.relu(dinv * za + ab2[...])
    mean_o[...] = jnp.tanh(
        jnp.dot(ha, mW[...], preferred_element_type=_f32) + mb[...])
    std_o[...] = jnp.broadcast_to(jnp.exp(lstd[...]), (BM, 8))

    zc = jnp.concatenate([acl[...] + ycl[...], ach[...] + ych[...]], axis=1)
    hc = jax.nn.relu(dinv * zc + cb2[...])
    b = batch_ref[...][:, 0:1]
    cols = lax.broadcasted_iota(jnp.int32, (BM, G), 1)
    oh = (b == cols).astype(_f32)

    @pl.when(i == 0)
    def _():
        sums_sc[...] = jnp.zeros_like(sums_sc)
        counts_sc[...] = jnp.zeros_like(counts_sc)

    dn = (((0,), (0,)), ((), ()))
    sums_sc[...] += lax.dot_general(oh, hc, dn, preferred_element_type=_f32)
    counts_sc[...] += lax.dot_general(
        oh, jnp.ones((BM, 8), _f32), dn, preferred_element_type=_f32)

    cnt = jnp.maximum(counts_sc[:, 0:1], 1.0)
    gm = sums_sc[...] / cnt
    hid = jax.nn.relu(jnp.dot(gm, fW1[...], preferred_element_type=_f32) + fb1[...])
    val_o[...] = jnp.dot(hid, fW2[...], preferred_element_type=_f32) + fb2[...]


_head = pl.pallas_call(
    _head_body,
    grid=(GRID,),
    in_specs=[pl.BlockSpec((BM, H), lambda i: (i, 0))] * 8 + [
        pl.BlockSpec((2, BM, 16), lambda i: (0, i, 0)),
        pl.BlockSpec((BM, 8), lambda i: (i, 0)),
        pl.BlockSpec((1, D), lambda i: (0, 0)),
        pl.BlockSpec((1, D), lambda i: (0, 0)),
        pl.BlockSpec((D, 8), lambda i: (0, 0)),
        pl.BlockSpec((1, 8), lambda i: (0, 0)),
        pl.BlockSpec((1, 8), lambda i: (0, 0)),
        pl.BlockSpec((D, G), lambda i: (0, 0)),
        pl.BlockSpec((1, G), lambda i: (0, 0)),
        pl.BlockSpec((G, 1), lambda i: (0, 0)),
        pl.BlockSpec((1, 1), lambda i: (0, 0)),
    ],
    out_specs=[
        pl.BlockSpec((BM, 8), lambda i: (i, 0)),
        pl.BlockSpec((BM, 8), lambda i: (i, 0)),
        pl.BlockSpec((G, 1), lambda i: (0, 0)),
    ],
    out_shape=[
        jax.ShapeDtypeStruct((NP, 8), _f32),
        jax.ShapeDtypeStruct((NP, 8), _f32),
        jax.ShapeDtypeStruct((G, 1), _f32),
    ],
    scratch_shapes=[
        pltpu.VMEM((G, D), _f32),
        pltpu.VMEM((G, 8), _f32),
    ],
)


def kernel(x, edge_index, batch, aW1, ab1, aW2, ab2, mW, mb, log_std,
           cW1, cb1, cW2, cb2, fW1, fb1, fW2, fb2):
    src = edge_index[0].astype(jnp.int32)
    dst = edge_index[1].astype(jnp.int32)
    pad_idx = N + (jnp.arange(EP - E, dtype=jnp.int32) % (NP - N))
    src_p = jnp.concatenate([src, pad_idx]).reshape(ROWS_E, 128)
    dst_p = jnp.concatenate([dst, pad_idx]).reshape(ROWS_E, 128)
    x_p = jnp.zeros((NP, D), _f32).at[:N].set(x)
    batch_p = jnp.full((NP, 8), G, jnp.int32).at[:N].set(
        jnp.broadcast_to(batch.astype(jnp.int32)[:, None], (N, 8)))
    W1 = jnp.concatenate([aW1, cW1], axis=1)

    deg2 = _deg_call(dst_p)
    y1al, y1ah, y1cl, y1ch = _prep(x_p, deg2, W1)
    a1al, a1cl = _edge_call(y1al, y1cl, src_p, dst_p)
    a1ah, a1ch = _edge_call(y1ah, y1ch, src_p, dst_p)
    y2al, y2ah, y2cl, y2ch = _mid(a1al, a1ah, a1cl, a1ch,
                                  y1al, y1ah, y1cl, y1ch, deg2, aW2, cW2,
                                  ab1.reshape(1, D), cb1.reshape(1, D))
    a2al, a2cl = _edge_call(y2al, y2cl, src_p, dst_p)
    a2ah, a2ch = _edge_call(y2ah, y2ch, src_p, dst_p)
    mean, std, value = _head(a2al, a2ah, a2cl, a2ch,
                             y2al, y2ah, y2cl, y2ch, deg2, batch_p,
                             ab2.reshape(1, D), cb2.reshape(1, D),
                             mW, mb.reshape(1, 8), log_std,
                             fW1, fb1.reshape(1, G), fW2, fb2.reshape(1, 1))
    return (mean[:N], std[:N], value)
